# baseline (speedup 1.0000x reference)
import functools

import jax
import jax.numpy as jnp
from jax import lax
from jax.experimental import pallas as pl
from jax.experimental.pallas import tpu as pltpu
from jax.experimental.pallas import tpu_sc as plsc

H = 128
D = 64
W = 128

BLK_E = 2000
BLK_T = 4000

NCORE = 2
NSUB = 16
NCHUNK = 50
SUB = 32
ZO = 40


def _silu(v):
    return v * (1.0 / (1.0 + jnp.exp(-v)))


def _edge_down_body(x_ref, rbf_ref, w_rbf1, w_rbf2, w_kj, b_kj, w_down, o_ref):
    x = x_ref[...]
    xk = _silu(jnp.dot(x, w_kj[...], preferred_element_type=jnp.float32) + b_kj[...])
    rbf_e = jnp.dot(
        jnp.dot(rbf_ref[...], w_rbf1[...], preferred_element_type=jnp.float32),
        w_rbf2[...], preferred_element_type=jnp.float32)
    val = _silu(jnp.dot(xk * rbf_e, w_down[...],
                        preferred_element_type=jnp.float32))
    o_ref[...] = jnp.concatenate([val, jnp.zeros_like(val)], axis=1)


def _edge_down(x, rbf, w_rbf1, w_rbf2, w_kj, b_kj, w_down):
    e = x.shape[0]
    nr = rbf.shape[1]
    bas = w_rbf1.shape[1]
    grid = e // BLK_E
    full = lambda *shape: pl.BlockSpec(shape, lambda i: (0,) * len(shape))
    return pl.pallas_call(
        _edge_down_body,
        grid=(grid,),
        in_specs=[
            pl.BlockSpec((BLK_E, H), lambda i: (i, 0)),
            pl.BlockSpec((BLK_E, nr), lambda i: (i, 0)),
            full(nr, bas), full(bas, H), full(H, H), full(1, H), full(H, D),
        ],
        out_specs=pl.BlockSpec((BLK_E, W), lambda i: (i, 0)),
        out_shape=jax.ShapeDtypeStruct((e, W), jnp.float32),
    )(x, rbf, w_rbf1, w_rbf2, w_kj, b_kj, w_down)


def _sbf_body(sbf_ref, w_sbf1, w_sbf2, o_ref):
    val = jnp.dot(
        jnp.dot(sbf_ref[...], w_sbf1[...], preferred_element_type=jnp.float32),
        w_sbf2[...], preferred_element_type=jnp.float32)
    o_ref[...] = jnp.concatenate([val, jnp.zeros_like(val)], axis=1)


def _sbf_emb(sbf, w_sbf1, w_sbf2):
    t, sr = sbf.shape
    bas = w_sbf1.shape[1]
    grid = t // BLK_T
    full = lambda *shape: pl.BlockSpec(shape, lambda i: (0,) * len(shape))
    return pl.pallas_call(
        _sbf_body,
        grid=(grid,),
        in_specs=[
            pl.BlockSpec((BLK_T, sr), lambda i: (i, 0)),
            full(sr, bas), full(bas, D),
        ],
        out_specs=pl.BlockSpec((BLK_T, W), lambda i: (i, 0)),
        out_shape=jax.ShapeDtypeStruct((t, W), jnp.float32),
    )(sbf, w_sbf1, w_sbf2)


def _make_sc_scatter(e, t):
    chunk = e // NCHUNK
    cpc = NCHUNK // NCORE
    rps = chunk // NSUB
    slice_t = t // NSUB
    sb = 4000
    nb_scan = slice_t // sb
    listcap = sb + SUB + 32

    mesh = plsc.VectorSubcoreMesh(core_axis_name="c", subcore_axis_name="s",
                                  num_cores=NCORE, num_subcores=NSUB)

    @functools.partial(
        pl.kernel, mesh=mesh,
        compiler_params=pltpu.CompilerParams(needs_layout_passes=False),
        out_type=jax.ShapeDtypeStruct((e, W), jnp.float32),
        scratch_types=[
            pltpu.VMEM((sb,), jnp.int32),
            pltpu.VMEM((sb,), jnp.int32),
            pltpu.VMEM((listcap,), jnp.int32),
            pltpu.VMEM((listcap,), jnp.int32),
            pltpu.VMEM((SUB,), jnp.int32),
            pltpu.VMEM((SUB,), jnp.int32),
            pltpu.VMEM((SUB,), jnp.int32),
            pltpu.VMEM((SUB, W), jnp.float32),
            pltpu.VMEM((SUB, W), jnp.float32),
            pltpu.VMEM((SUB, W), jnp.float32),
            pltpu.VMEM((ZO, W), jnp.float32),
            pltpu.VMEM_SHARED((chunk + 8, W), jnp.float32),
            pltpu.SemaphoreType.DMA,
            pltpu.SemaphoreType.DMA,
        ],
    )
    def sc_scatter(xkjd, sbfe, idxkj, idxji, out,
                   ji_buf, kj_buf, dst_list, t_list,
                   dst_stage, kj_stage, t_stage, rows_a, rows_b, prod,
                   zo, acc, sem_a, sem_b):
        c = lax.axis_index("c")
        s = lax.axis_index("s")

        def chunk_body(k, _):
            lo = (c * cpc + k) * chunk

            def zfill(i, _):
                zo[i // 8, pl.ds((i % 8) * 16, 16)] = jnp.zeros((16,),
                                                               jnp.float32)
                return 0
            lax.fori_loop(0, ZO * 8, zfill, 0)

            def zacc(i, _):
                pltpu.sync_copy(zo, acc.at[pl.ds(s * rps + i * ZO, ZO)])
                return 0
            lax.fori_loop(0, rps // ZO, zacc, 0)
            plsc.subcore_barrier()

            def batch_body(b, _):
                base = s * slice_t + b * sb
                pltpu.sync_copy(idxji.at[pl.ds(base, sb)], ji_buf)
                pltpu.sync_copy(idxkj.at[pl.ds(base, sb)], kj_buf)

                def scan_body(v, cnt):
                    ji = ji_buf[pl.ds(v * 16, 16)]
                    lov = jnp.full((16,), lo, jnp.int32)
                    m = (ji >= lov) & (ji < lov + chunk)
                    mi = m.astype(jnp.int32)
                    pos = plsc.cumsum(mi) + jnp.full((16,), cnt - 1, jnp.int32)
                    tg = lax.iota(jnp.int32, 16) + jnp.full(
                        (16,), base + v * 16, jnp.int32)
                    plsc.store_scatter(dst_list, [pos], ji - lov, mask=m)
                    plsc.store_scatter(t_list, [pos], tg, mask=m)
                    cntv = plsc.all_reduce_population_count(m)
                    return cnt + cntv[0]
                cnt = lax.fori_loop(0, sb // 16, scan_body, jnp.int32(0))

                for w in range(SUB // 16):
                    sl = pl.ds(cnt + w * 16, 16)
                    dst_list[sl] = jnp.full((16,), chunk, jnp.int32)
                    t_list[sl] = jnp.full((16,), base, jnp.int32)
                nsb = (cnt + SUB - 1) // SUB

                def drain(j, _):
                    off = j * SUB
                    basev = jnp.full((16,), base, jnp.int32)
                    for w in range(SUB // 16):
                        sl = pl.ds(w * 16, 16)
                        tg = t_list[pl.ds(off + w * 16, 16)]
                        dst_stage[sl] = dst_list[pl.ds(off + w * 16, 16)]
                        t_stage[sl] = tg
                        kj_stage[sl] = plsc.load_gather(kj_buf, [tg - basev])
                    da = pltpu.async_copy(xkjd.at[kj_stage], rows_a, sem_a)
                    db = pltpu.async_copy(sbfe.at[t_stage], rows_b, sem_b)
                    da.wait()
                    db.wait()

                    def mul(r, _):
                        for q in range(W // 16):
                            sl = pl.ds(q * 16, 16)
                            prod[r, sl] = rows_a[r, sl] * rows_b[r, sl]
                        return 0
                    lax.fori_loop(0, SUB, mul, 0)
                    pltpu.sync_copy(prod, acc.at[dst_stage], add=True)
                    return 0
                lax.fori_loop(0, nsb, drain, 0)
                return 0
            lax.fori_loop(0, nb_scan, batch_body, 0)
            plsc.subcore_barrier()

            def wout(i, _):
                r0 = s * rps + i * ZO
                pltpu.sync_copy(acc.at[pl.ds(r0, ZO)], zo)
                pltpu.sync_copy(zo, out.at[pl.ds(lo + r0, ZO)])
                return 0
            lax.fori_loop(0, rps // ZO, wout, 0)
            plsc.subcore_barrier()
            return 0
        lax.fori_loop(0, cpc, chunk_body, 0)

    return sc_scatter


def _tail_body(x_ref, spe_ref, alpha_ref, w_ji, b_ji, w_up,
               rb1_w1, rb1_b1, rb1_w2, rb1_b2, w_lin, b_lin,
               ra1_w1, ra1_b1, ra1_w2, ra1_b2, o_ref):
    x = x_ref[...]
    x_ji = _silu(jnp.dot(x, w_ji[...], preferred_element_type=jnp.float32)
                 + b_ji[...])
    spe = spe_ref[...][:, :D] * alpha_ref[0, 0]
    x_up = _silu(jnp.dot(spe, w_up[...], preferred_element_type=jnp.float32))
    h = x_ji + x_up
    h1 = _silu(jnp.dot(h, rb1_w1[...], preferred_element_type=jnp.float32)
               + rb1_b1[...])
    h = h + _silu(jnp.dot(h1, rb1_w2[...], preferred_element_type=jnp.float32)
                  + rb1_b2[...])
    h = _silu(jnp.dot(h, w_lin[...], preferred_element_type=jnp.float32)
              + b_lin[...]) + x
    h2 = _silu(jnp.dot(h, ra1_w1[...], preferred_element_type=jnp.float32)
               + ra1_b1[...])
    o_ref[...] = h + _silu(jnp.dot(h2, ra1_w2[...],
                                   preferred_element_type=jnp.float32)
                           + ra1_b2[...])


def _tail(x, spe, alpha_arr, w_ji, b_ji, w_up, rb1_w1, rb1_b1, rb1_w2, rb1_b2,
          w_lin, b_lin, ra1_w1, ra1_b1, ra1_w2, ra1_b2):
    e = x.shape[0]
    grid = e // BLK_E
    full = lambda *shape: pl.BlockSpec(shape, lambda i: (0,) * len(shape))
    return pl.pallas_call(
        _tail_body,
        grid=(grid,),
        in_specs=[
            pl.BlockSpec((BLK_E, H), lambda i: (i, 0)),
            pl.BlockSpec((BLK_E, W), lambda i: (i, 0)),
            full(1, 1),
            full(H, H), full(1, H), full(D, H),
            full(H, H), full(1, H), full(H, H), full(1, H),
            full(H, H), full(1, H),
            full(H, H), full(1, H), full(H, H), full(1, H),
        ],
        out_specs=pl.BlockSpec((BLK_E, H), lambda i: (i, 0)),
        out_shape=jax.ShapeDtypeStruct((e, H), jnp.float32),
    )(x, spe, alpha_arr, w_ji, b_ji, w_up, rb1_w1, rb1_b1, rb1_w2, rb1_b2,
      w_lin, b_lin, ra1_w1, ra1_b1, ra1_w2, ra1_b2)


def kernel(x, rbf, sbf, idx_kj, idx_ji, bt, lambda_d, alpha,
           w_rbf1, w_rbf2, w_sbf1, w_sbf2, w_kj, b_kj, w_ji, b_ji,
           w_down, w_up, rb1_w1, rb1_b1, rb1_w2, rb1_b2,
           w_lin, b_lin, ra1_w1, ra1_b1, ra1_w2, ra1_b2):
    e = x.shape[0]
    t = sbf.shape[0]

    xkjd = _edge_down(x, rbf, w_rbf1, w_rbf2, w_kj,
                      b_kj.reshape(1, H), w_down)
    sbfe = _sbf_emb(sbf, w_sbf1, w_sbf2)

    sc_scatter = _make_sc_scatter(e, t)
    spe = sc_scatter(xkjd, sbfe,
                     idx_kj.astype(jnp.int32), idx_ji.astype(jnp.int32))

    alpha_arr = jnp.asarray(alpha, jnp.float32).reshape(1, 1)
    return _tail(x, spe, alpha_arr, w_ji, b_ji.reshape(1, H), w_up,
                 rb1_w1, rb1_b1.reshape(1, H), rb1_w2, rb1_b2.reshape(1, H),
                 w_lin, b_lin.reshape(1, H),
                 ra1_w1, ra1_b1.reshape(1, H), ra1_w2, ra1_b2.reshape(1, H))


# speedup vs baseline: 1.0826x; 1.0826x over previous
import functools

import jax
import jax.numpy as jnp
from jax import lax
from jax.experimental import pallas as pl
from jax.experimental.pallas import tpu as pltpu
from jax.experimental.pallas import tpu_sc as plsc

H = 128
D = 64
W = 128

BLK_E = 2000
BLK_T = 4000

NCORE = 2
NSUB = 16
NCHUNK = 50
SUB = 32
ZO = 40


def _silu(v):
    return v * (1.0 / (1.0 + jnp.exp(-v)))


def _edge_down_body(x_ref, rbf_ref, w_rbf1, w_rbf2, w_kj, b_kj, w_down, o_ref):
    x = x_ref[...]
    xk = _silu(jnp.dot(x, w_kj[...], preferred_element_type=jnp.float32) + b_kj[...])
    rbf_e = jnp.dot(
        jnp.dot(rbf_ref[...], w_rbf1[...], preferred_element_type=jnp.float32),
        w_rbf2[...], preferred_element_type=jnp.float32)
    val = _silu(jnp.dot(xk * rbf_e, w_down[...],
                        preferred_element_type=jnp.float32))
    o_ref[...] = jnp.concatenate([val, jnp.zeros_like(val)], axis=1)


def _edge_down(x, rbf, w_rbf1, w_rbf2, w_kj, b_kj, w_down):
    e = x.shape[0]
    nr = rbf.shape[1]
    bas = w_rbf1.shape[1]
    grid = e // BLK_E
    full = lambda *shape: pl.BlockSpec(shape, lambda i: (0,) * len(shape))
    return pl.pallas_call(
        _edge_down_body,
        grid=(grid,),
        in_specs=[
            pl.BlockSpec((BLK_E, H), lambda i: (i, 0)),
            pl.BlockSpec((BLK_E, nr), lambda i: (i, 0)),
            full(nr, bas), full(bas, H), full(H, H), full(1, H), full(H, D),
        ],
        out_specs=pl.BlockSpec((BLK_E, W), lambda i: (i, 0)),
        out_shape=jax.ShapeDtypeStruct((e, W), jnp.float32),
    )(x, rbf, w_rbf1, w_rbf2, w_kj, b_kj, w_down)


def _sbf_body(sbf_ref, w_sbf1, w_sbf2, o_ref):
    val = jnp.dot(
        jnp.dot(sbf_ref[...], w_sbf1[...], preferred_element_type=jnp.float32),
        w_sbf2[...], preferred_element_type=jnp.float32)
    o_ref[...] = jnp.concatenate([val, jnp.zeros_like(val)], axis=1)


def _sbf_emb(sbf, w_sbf1, w_sbf2):
    t, sr = sbf.shape
    bas = w_sbf1.shape[1]
    grid = t // BLK_T
    full = lambda *shape: pl.BlockSpec(shape, lambda i: (0,) * len(shape))
    return pl.pallas_call(
        _sbf_body,
        grid=(grid,),
        in_specs=[
            pl.BlockSpec((BLK_T, sr), lambda i: (i, 0)),
            full(sr, bas), full(bas, D),
        ],
        out_specs=pl.BlockSpec((BLK_T, W), lambda i: (i, 0)),
        out_shape=jax.ShapeDtypeStruct((t, W), jnp.float32),
    )(sbf, w_sbf1, w_sbf2)


def _make_sc_scatter(e, t):
    chunk = e // NCHUNK
    cpc = NCHUNK // NCORE
    rps = chunk // NSUB
    slice_t = t // NSUB
    sb = 4000
    nb_scan = slice_t // sb
    listcap = sb + SUB + 32

    mesh = plsc.VectorSubcoreMesh(core_axis_name="c", subcore_axis_name="s",
                                  num_cores=NCORE, num_subcores=NSUB)

    @functools.partial(
        pl.kernel, mesh=mesh,
        compiler_params=pltpu.CompilerParams(needs_layout_passes=False),
        out_type=jax.ShapeDtypeStruct((e, W), jnp.float32),
        scratch_types=[
            pltpu.VMEM((sb,), jnp.int32),
            pltpu.VMEM((sb,), jnp.int32),
            pltpu.VMEM((listcap,), jnp.int32),
            pltpu.VMEM((listcap,), jnp.int32),
            [pltpu.VMEM((SUB,), jnp.int32)] * 2,
            [pltpu.VMEM((SUB,), jnp.int32)] * 2,
            [pltpu.VMEM((SUB,), jnp.int32)] * 2,
            [pltpu.VMEM((SUB, W), jnp.float32)] * 2,
            [pltpu.VMEM((SUB, W), jnp.float32)] * 2,
            pltpu.VMEM((SUB, W), jnp.float32),
            pltpu.VMEM((ZO, W), jnp.float32),
            pltpu.VMEM_SHARED((chunk + 8, W), jnp.float32),
            [pltpu.SemaphoreType.DMA] * 2,
            [pltpu.SemaphoreType.DMA] * 2,
        ],
    )
    def sc_scatter(xkjd, sbfe, idxkj, idxji, out,
                   ji_buf, kj_buf, dst_list, t_list,
                   dst_stage, kj_stage, t_stage, rows_a, rows_b, prod,
                   zo, acc, sem_a, sem_b):
        c = lax.axis_index("c")
        s = lax.axis_index("s")

        def chunk_body(k, _):
            lo = (c * cpc + k) * chunk

            def zfill(i, _):
                zo[i // 8, pl.ds((i % 8) * 16, 16)] = jnp.zeros((16,),
                                                               jnp.float32)
                return 0
            lax.fori_loop(0, ZO * 8, zfill, 0)

            def zacc(i, _):
                pltpu.sync_copy(zo, acc.at[pl.ds(s * rps + i * ZO, ZO)])
                return 0
            lax.fori_loop(0, rps // ZO, zacc, 0)
            plsc.subcore_barrier()

            def batch_body(b, _):
                base = s * slice_t + b * sb
                pltpu.sync_copy(idxji.at[pl.ds(base, sb)], ji_buf)
                pltpu.sync_copy(idxkj.at[pl.ds(base, sb)], kj_buf)

                def scan_body(v, cnt):
                    ji = ji_buf[pl.ds(v * 16, 16)]
                    lov = jnp.full((16,), lo, jnp.int32)
                    m = (ji >= lov) & (ji < lov + chunk)
                    mi = m.astype(jnp.int32)
                    pos = plsc.cumsum(mi) + jnp.full((16,), cnt - 1, jnp.int32)
                    tg = lax.iota(jnp.int32, 16) + jnp.full(
                        (16,), base + v * 16, jnp.int32)
                    plsc.store_scatter(dst_list, [pos], ji - lov, mask=m)
                    plsc.store_scatter(t_list, [pos], tg, mask=m)
                    cntv = plsc.all_reduce_population_count(m)
                    return cnt + cntv[0]
                cnt = lax.fori_loop(0, sb // 16, scan_body, jnp.int32(0))

                for w in range(SUB // 16):
                    sl = pl.ds(cnt + w * 16, 16)
                    dst_list[sl] = jnp.full((16,), chunk, jnp.int32)
                    t_list[sl] = jnp.full((16,), base, jnp.int32)
                nsb = (cnt + SUB - 1) // SUB
                basev = jnp.full((16,), base, jnp.int32)

                def gstage(j, u):
                    off = j * SUB
                    for w in range(SUB // 16):
                        sl = pl.ds(w * 16, 16)
                        tg = t_list[pl.ds(off + w * 16, 16)]
                        dst_stage[u][sl] = dst_list[pl.ds(off + w * 16, 16)]
                        t_stage[u][sl] = tg
                        kj_stage[u][sl] = plsc.load_gather(kj_buf,
                                                           [tg - basev])
                    pltpu.async_copy(xkjd.at[kj_stage[u]], rows_a[u],
                                     sem_a[u])
                    pltpu.async_copy(sbfe.at[t_stage[u]], rows_b[u],
                                     sem_b[u])

                def consume(u):
                    pltpu.make_async_copy(xkjd.at[kj_stage[u]], rows_a[u],
                                          sem_a[u]).wait()
                    pltpu.make_async_copy(sbfe.at[t_stage[u]], rows_b[u],
                                          sem_b[u]).wait()

                    def mul(r, _):
                        for q in range(W // 16):
                            sl = pl.ds(q * 16, 16)
                            prod[r, sl] = rows_a[u][r, sl] * rows_b[u][r, sl]
                        return 0
                    lax.fori_loop(0, SUB, mul, 0)
                    pltpu.sync_copy(prod, acc.at[dst_stage[u]], add=True)

                @pl.when(nsb > 0)
                def _():
                    gstage(jnp.int32(0), 0)

                def drain_pair(p, _):
                    j0 = p * 2
                    j1 = j0 + 1

                    @pl.when(j1 < nsb)
                    def _():
                        gstage(j1, 1)

                    @pl.when(j0 < nsb)
                    def _():
                        consume(0)

                    @pl.when(j0 + 2 < nsb)
                    def _():
                        gstage(j0 + 2, 0)

                    @pl.when(j1 < nsb)
                    def _():
                        consume(1)
                    return 0
                lax.fori_loop(0, (nsb + 1) // 2, drain_pair, 0)
                return 0
            lax.fori_loop(0, nb_scan, batch_body, 0)
            plsc.subcore_barrier()

            def wout(i, _):
                r0 = s * rps + i * ZO
                pltpu.sync_copy(acc.at[pl.ds(r0, ZO)], zo)
                pltpu.sync_copy(zo, out.at[pl.ds(lo + r0, ZO)])
                return 0
            lax.fori_loop(0, rps // ZO, wout, 0)
            plsc.subcore_barrier()
            return 0
        lax.fori_loop(0, cpc, chunk_body, 0)

    return sc_scatter


def _tail_body(x_ref, spe_ref, alpha_ref, w_ji, b_ji, w_up,
               rb1_w1, rb1_b1, rb1_w2, rb1_b2, w_lin, b_lin,
               ra1_w1, ra1_b1, ra1_w2, ra1_b2, o_ref):
    x = x_ref[...]
    x_ji = _silu(jnp.dot(x, w_ji[...], preferred_element_type=jnp.float32)
                 + b_ji[...])
    spe = spe_ref[...][:, :D] * alpha_ref[0, 0]
    x_up = _silu(jnp.dot(spe, w_up[...], preferred_element_type=jnp.float32))
    h = x_ji + x_up
    h1 = _silu(jnp.dot(h, rb1_w1[...], preferred_element_type=jnp.float32)
               + rb1_b1[...])
    h = h + _silu(jnp.dot(h1, rb1_w2[...], preferred_element_type=jnp.float32)
                  + rb1_b2[...])
    h = _silu(jnp.dot(h, w_lin[...], preferred_element_type=jnp.float32)
              + b_lin[...]) + x
    h2 = _silu(jnp.dot(h, ra1_w1[...], preferred_element_type=jnp.float32)
               + ra1_b1[...])
    o_ref[...] = h + _silu(jnp.dot(h2, ra1_w2[...],
                                   preferred_element_type=jnp.float32)
                           + ra1_b2[...])


def _tail(x, spe, alpha_arr, w_ji, b_ji, w_up, rb1_w1, rb1_b1, rb1_w2, rb1_b2,
          w_lin, b_lin, ra1_w1, ra1_b1, ra1_w2, ra1_b2):
    e = x.shape[0]
    grid = e // BLK_E
    full = lambda *shape: pl.BlockSpec(shape, lambda i: (0,) * len(shape))
    return pl.pallas_call(
        _tail_body,
        grid=(grid,),
        in_specs=[
            pl.BlockSpec((BLK_E, H), lambda i: (i, 0)),
            pl.BlockSpec((BLK_E, W), lambda i: (i, 0)),
            full(1, 1),
            full(H, H), full(1, H), full(D, H),
            full(H, H), full(1, H), full(H, H), full(1, H),
            full(H, H), full(1, H),
            full(H, H), full(1, H), full(H, H), full(1, H),
        ],
        out_specs=pl.BlockSpec((BLK_E, H), lambda i: (i, 0)),
        out_shape=jax.ShapeDtypeStruct((e, H), jnp.float32),
    )(x, spe, alpha_arr, w_ji, b_ji, w_up, rb1_w1, rb1_b1, rb1_w2, rb1_b2,
      w_lin, b_lin, ra1_w1, ra1_b1, ra1_w2, ra1_b2)


def kernel(x, rbf, sbf, idx_kj, idx_ji, bt, lambda_d, alpha,
           w_rbf1, w_rbf2, w_sbf1, w_sbf2, w_kj, b_kj, w_ji, b_ji,
           w_down, w_up, rb1_w1, rb1_b1, rb1_w2, rb1_b2,
           w_lin, b_lin, ra1_w1, ra1_b1, ra1_w2, ra1_b2):
    e = x.shape[0]
    t = sbf.shape[0]

    xkjd = _edge_down(x, rbf, w_rbf1, w_rbf2, w_kj,
                      b_kj.reshape(1, H), w_down)
    sbfe = _sbf_emb(sbf, w_sbf1, w_sbf2)

    sc_scatter = _make_sc_scatter(e, t)
    spe = sc_scatter(xkjd, sbfe,
                     idx_kj.astype(jnp.int32), idx_ji.astype(jnp.int32))

    alpha_arr = jnp.asarray(alpha, jnp.float32).reshape(1, 1)
    return _tail(x, spe, alpha_arr, w_ji, b_ji.reshape(1, H), w_up,
                 rb1_w1, rb1_b1.reshape(1, H), rb1_w2, rb1_b2.reshape(1, H),
                 w_lin, b_lin.reshape(1, H),
                 ra1_w1, ra1_b1.reshape(1, H), ra1_w2, ra1_b2.reshape(1, H))


# speedup vs baseline: 1.0944x; 1.0109x over previous
import functools

import jax
import jax.numpy as jnp
from jax import lax
from jax.experimental import pallas as pl
from jax.experimental.pallas import tpu as pltpu
from jax.experimental.pallas import tpu_sc as plsc

H = 128
D = 64
W = 128

BLK_E = 2000
BLK_T = 4000

NCORE = 2
NSUB = 16
NCHUNK = 50
SUB = 32
ZO = 40


def _silu(v):
    return v * (1.0 / (1.0 + jnp.exp(-v)))


def _edge_down_body(x_ref, rbf_ref, w_rbf1, w_rbf2, w_kj, b_kj, w_down, o_ref):
    x = x_ref[...]
    xk = _silu(jnp.dot(x, w_kj[...], preferred_element_type=jnp.float32) + b_kj[...])
    rbf_e = jnp.dot(
        jnp.dot(rbf_ref[...], w_rbf1[...], preferred_element_type=jnp.float32),
        w_rbf2[...], preferred_element_type=jnp.float32)
    val = _silu(jnp.dot(xk * rbf_e, w_down[...],
                        preferred_element_type=jnp.float32))
    o_ref[...] = jnp.concatenate([val, jnp.zeros_like(val)], axis=1)


def _edge_down(x, rbf, w_rbf1, w_rbf2, w_kj, b_kj, w_down):
    e = x.shape[0]
    nr = rbf.shape[1]
    bas = w_rbf1.shape[1]
    grid = e // BLK_E
    full = lambda *shape: pl.BlockSpec(shape, lambda i: (0,) * len(shape))
    return pl.pallas_call(
        _edge_down_body,
        grid=(grid,),
        in_specs=[
            pl.BlockSpec((BLK_E, H), lambda i: (i, 0)),
            pl.BlockSpec((BLK_E, nr), lambda i: (i, 0)),
            full(nr, bas), full(bas, H), full(H, H), full(1, H), full(H, D),
        ],
        out_specs=pl.BlockSpec((BLK_E, W), lambda i: (i, 0)),
        out_shape=jax.ShapeDtypeStruct((e, W), jnp.float32),
    )(x, rbf, w_rbf1, w_rbf2, w_kj, b_kj, w_down)


def _sbf_body(sbf_ref, w_sbf1, w_sbf2, o_ref):
    val = jnp.dot(
        jnp.dot(sbf_ref[...], w_sbf1[...], preferred_element_type=jnp.float32),
        w_sbf2[...], preferred_element_type=jnp.float32)
    o_ref[...] = jnp.concatenate([val, jnp.zeros_like(val)], axis=1)


def _sbf_emb(sbf, w_sbf1, w_sbf2):
    t, sr = sbf.shape
    bas = w_sbf1.shape[1]
    grid = t // BLK_T
    full = lambda *shape: pl.BlockSpec(shape, lambda i: (0,) * len(shape))
    return pl.pallas_call(
        _sbf_body,
        grid=(grid,),
        in_specs=[
            pl.BlockSpec((BLK_T, sr), lambda i: (i, 0)),
            full(sr, bas), full(bas, D),
        ],
        out_specs=pl.BlockSpec((BLK_T, W), lambda i: (i, 0)),
        out_shape=jax.ShapeDtypeStruct((t, W), jnp.float32),
    )(sbf, w_sbf1, w_sbf2)


def _make_sc_scatter(e, t):
    chunk = e // NCHUNK
    cpc = NCHUNK // NCORE
    rps = chunk // NSUB
    slice_t = t // NSUB
    sb = 4000
    nb_scan = slice_t // sb
    listcap = sb + SUB + 32

    mesh = plsc.VectorSubcoreMesh(core_axis_name="c", subcore_axis_name="s",
                                  num_cores=NCORE, num_subcores=NSUB)

    @functools.partial(
        pl.kernel, mesh=mesh,
        compiler_params=pltpu.CompilerParams(needs_layout_passes=False),
        out_type=jax.ShapeDtypeStruct((e, W), jnp.float32),
        scratch_types=[
            pltpu.VMEM((sb,), jnp.int32),
            pltpu.VMEM((sb,), jnp.int32),
            pltpu.VMEM((listcap,), jnp.int32),
            pltpu.VMEM((listcap,), jnp.int32),
            [pltpu.VMEM((SUB,), jnp.int32)] * 2,
            [pltpu.VMEM((SUB,), jnp.int32)] * 2,
            [pltpu.VMEM((SUB,), jnp.int32)] * 2,
            [pltpu.VMEM((SUB, W), jnp.float32)] * 2,
            [pltpu.VMEM((SUB, W), jnp.float32)] * 2,
            pltpu.VMEM((SUB, W), jnp.float32),
            pltpu.VMEM((ZO, W), jnp.float32),
            pltpu.VMEM_SHARED((chunk + 8, W), jnp.float32),
            [pltpu.SemaphoreType.DMA] * 2,
            [pltpu.SemaphoreType.DMA] * 2,
        ],
    )
    def sc_scatter(xkjd, sbfe, idxkj, idxji, out,
                   ji_buf, kj_buf, dst_list, t_list,
                   dst_stage, kj_stage, t_stage, rows_a, rows_b, prod,
                   zo, acc, sem_a, sem_b):
        c = lax.axis_index("c")
        s = lax.axis_index("s")

        def chunk_body(k, _):
            lo = (c * cpc + k) * chunk

            def zfill(i, _):
                zo[i // 8, pl.ds((i % 8) * 16, 16)] = jnp.zeros((16,),
                                                               jnp.float32)
                return 0
            lax.fori_loop(0, ZO * 8, zfill, 0)

            def zacc(i, _):
                pltpu.sync_copy(zo, acc.at[pl.ds(s * rps + i * ZO, ZO)])
                return 0
            lax.fori_loop(0, rps // ZO, zacc, 0)
            plsc.subcore_barrier()

            def batch_body(b, _):
                base = s * slice_t + b * sb
                pltpu.sync_copy(idxji.at[pl.ds(base, sb)], ji_buf)
                pltpu.sync_copy(idxkj.at[pl.ds(base, sb)], kj_buf)

                def scan_body(i, cnt):
                    lov = jnp.full((16,), lo, jnp.int32)
                    for u in range(5):
                        v = i * 5 + u
                        ji = ji_buf[pl.ds(v * 16, 16)]
                        m = (ji >= lov) & (ji < lov + chunk)
                        mi = m.astype(jnp.int32)
                        pos = plsc.cumsum(mi) + jnp.full((16,), cnt - 1,
                                                         jnp.int32)
                        tg = lax.iota(jnp.int32, 16) + jnp.full(
                            (16,), base + v * 16, jnp.int32)
                        plsc.store_scatter(dst_list, [pos], ji - lov, mask=m)
                        plsc.store_scatter(t_list, [pos], tg, mask=m)
                        cnt = cnt + plsc.all_reduce_population_count(m)[0]
                    return cnt
                cnt = lax.fori_loop(0, sb // 80, scan_body, jnp.int32(0))

                for w in range(SUB // 16):
                    sl = pl.ds(cnt + w * 16, 16)
                    dst_list[sl] = jnp.full((16,), chunk, jnp.int32)
                    t_list[sl] = jnp.full((16,), base, jnp.int32)
                nsb = (cnt + SUB - 1) // SUB
                basev = jnp.full((16,), base, jnp.int32)

                def gstage(j, u):
                    off = j * SUB
                    for w in range(SUB // 16):
                        sl = pl.ds(w * 16, 16)
                        tg = t_list[pl.ds(off + w * 16, 16)]
                        dst_stage[u][sl] = dst_list[pl.ds(off + w * 16, 16)]
                        t_stage[u][sl] = tg
                        kj_stage[u][sl] = plsc.load_gather(kj_buf,
                                                           [tg - basev])
                    pltpu.async_copy(xkjd.at[kj_stage[u]], rows_a[u],
                                     sem_a[u])
                    pltpu.async_copy(sbfe.at[t_stage[u]], rows_b[u],
                                     sem_b[u])

                def consume(u):
                    pltpu.make_async_copy(xkjd.at[kj_stage[u]], rows_a[u],
                                          sem_a[u]).wait()
                    pltpu.make_async_copy(sbfe.at[t_stage[u]], rows_b[u],
                                          sem_b[u]).wait()

                    def mul(r, _):
                        for q in range(W // 16):
                            sl = pl.ds(q * 16, 16)
                            prod[r, sl] = rows_a[u][r, sl] * rows_b[u][r, sl]
                        return 0
                    lax.fori_loop(0, SUB, mul, 0)
                    pltpu.sync_copy(prod, acc.at[dst_stage[u]], add=True)

                @pl.when(nsb > 0)
                def _():
                    gstage(jnp.int32(0), 0)

                def drain_pair(p, _):
                    j0 = p * 2
                    j1 = j0 + 1

                    @pl.when(j1 < nsb)
                    def _():
                        gstage(j1, 1)

                    @pl.when(j0 < nsb)
                    def _():
                        consume(0)

                    @pl.when(j0 + 2 < nsb)
                    def _():
                        gstage(j0 + 2, 0)

                    @pl.when(j1 < nsb)
                    def _():
                        consume(1)
                    return 0
                lax.fori_loop(0, (nsb + 1) // 2, drain_pair, 0)
                return 0
            lax.fori_loop(0, nb_scan, batch_body, 0)
            plsc.subcore_barrier()

            def wout(i, _):
                r0 = s * rps + i * ZO
                pltpu.sync_copy(acc.at[pl.ds(r0, ZO)], zo)
                pltpu.sync_copy(zo, out.at[pl.ds(lo + r0, ZO)])
                return 0
            lax.fori_loop(0, rps // ZO, wout, 0)
            plsc.subcore_barrier()
            return 0
        lax.fori_loop(0, cpc, chunk_body, 0)

    return sc_scatter


def _tail_body(x_ref, spe_ref, alpha_ref, w_ji, b_ji, w_up,
               rb1_w1, rb1_b1, rb1_w2, rb1_b2, w_lin, b_lin,
               ra1_w1, ra1_b1, ra1_w2, ra1_b2, o_ref):
    x = x_ref[...]
    x_ji = _silu(jnp.dot(x, w_ji[...], preferred_element_type=jnp.float32)
                 + b_ji[...])
    spe = spe_ref[...][:, :D] * alpha_ref[0, 0]
    x_up = _silu(jnp.dot(spe, w_up[...], preferred_element_type=jnp.float32))
    h = x_ji + x_up
    h1 = _silu(jnp.dot(h, rb1_w1[...], preferred_element_type=jnp.float32)
               + rb1_b1[...])
    h = h + _silu(jnp.dot(h1, rb1_w2[...], preferred_element_type=jnp.float32)
                  + rb1_b2[...])
    h = _silu(jnp.dot(h, w_lin[...], preferred_element_type=jnp.float32)
              + b_lin[...]) + x
    h2 = _silu(jnp.dot(h, ra1_w1[...], preferred_element_type=jnp.float32)
               + ra1_b1[...])
    o_ref[...] = h + _silu(jnp.dot(h2, ra1_w2[...],
                                   preferred_element_type=jnp.float32)
                           + ra1_b2[...])


def _tail(x, spe, alpha_arr, w_ji, b_ji, w_up, rb1_w1, rb1_b1, rb1_w2, rb1_b2,
          w_lin, b_lin, ra1_w1, ra1_b1, ra1_w2, ra1_b2):
    e = x.shape[0]
    grid = e // BLK_E
    full = lambda *shape: pl.BlockSpec(shape, lambda i: (0,) * len(shape))
    return pl.pallas_call(
        _tail_body,
        grid=(grid,),
        in_specs=[
            pl.BlockSpec((BLK_E, H), lambda i: (i, 0)),
            pl.BlockSpec((BLK_E, W), lambda i: (i, 0)),
            full(1, 1),
            full(H, H), full(1, H), full(D, H),
            full(H, H), full(1, H), full(H, H), full(1, H),
            full(H, H), full(1, H),
            full(H, H), full(1, H), full(H, H), full(1, H),
        ],
        out_specs=pl.BlockSpec((BLK_E, H), lambda i: (i, 0)),
        out_shape=jax.ShapeDtypeStruct((e, H), jnp.float32),
    )(x, spe, alpha_arr, w_ji, b_ji, w_up, rb1_w1, rb1_b1, rb1_w2, rb1_b2,
      w_lin, b_lin, ra1_w1, ra1_b1, ra1_w2, ra1_b2)


def kernel(x, rbf, sbf, idx_kj, idx_ji, bt, lambda_d, alpha,
           w_rbf1, w_rbf2, w_sbf1, w_sbf2, w_kj, b_kj, w_ji, b_ji,
           w_down, w_up, rb1_w1, rb1_b1, rb1_w2, rb1_b2,
           w_lin, b_lin, ra1_w1, ra1_b1, ra1_w2, ra1_b2):
    e = x.shape[0]
    t = sbf.shape[0]

    xkjd = _edge_down(x, rbf, w_rbf1, w_rbf2, w_kj,
                      b_kj.reshape(1, H), w_down)
    sbfe = _sbf_emb(sbf, w_sbf1, w_sbf2)

    sc_scatter = _make_sc_scatter(e, t)
    spe = sc_scatter(xkjd, sbfe,
                     idx_kj.astype(jnp.int32), idx_ji.astype(jnp.int32))

    alpha_arr = jnp.asarray(alpha, jnp.float32).reshape(1, 1)
    return _tail(x, spe, alpha_arr, w_ji, b_ji.reshape(1, H), w_up,
                 rb1_w1, rb1_b1.reshape(1, H), rb1_w2, rb1_b2.reshape(1, H),
                 w_lin, b_lin.reshape(1, H),
                 ra1_w1, ra1_b1.reshape(1, H), ra1_w2, ra1_b2.reshape(1, H))
